# baseline (speedup 1.0000x reference)
import functools

import jax
import jax.numpy as jnp
from jax import lax
from jax.experimental import pallas as pl
from jax.experimental.pallas import tpu as pltpu
from jax.experimental.pallas import tpu_sc as plsc

_NC = 2
_NS = 16
_NW = _NC * _NS


@functools.partial(jax.jit, static_argnums=(2,))
def _gather(weight, flat_idx, chunk):
    B = flat_idx.shape[0]
    D = weight.shape[1]
    b_per_w = B // _NW
    n_chunks = b_per_w // chunk
    assert n_chunks % 2 == 0 and n_chunks * chunk == b_per_w
    mesh = plsc.VectorSubcoreMesh(core_axis_name="c", subcore_axis_name="s")

    @functools.partial(
        pl.kernel,
        mesh=mesh,
        out_type=jax.ShapeDtypeStruct((B, D), jnp.float32),
        scratch_types=[
            pltpu.VMEM((2, chunk), jnp.int32),
            pltpu.VMEM((2, chunk, D), jnp.float32),
            pltpu.SemaphoreType.DMA((2,)),
            pltpu.SemaphoreType.DMA((2,)),
            pltpu.SemaphoreType.DMA((2,)),
        ],
        compiler_params=pltpu.CompilerParams(use_tc_tiling_on_sc=False),
    )
    def k(table_hbm, idx_hbm, out_hbm, idx_v, rows_v, sem_i, sem_g, sem_s):
        wid = lax.axis_index("s") * _NC + lax.axis_index("c")
        base = wid * b_per_w

        def idx_copy(c, b):
            return pltpu.make_async_copy(
                idx_hbm.at[pl.ds(base + c * chunk, chunk)],
                idx_v.at[b], sem_i.at[b])

        def gather_copy(b):
            return pltpu.make_async_copy(
                table_hbm.at[idx_v.at[b]], rows_v.at[b], sem_g.at[b])

        def store_copy(c, b):
            return pltpu.make_async_copy(
                rows_v.at[b], out_hbm.at[pl.ds(base + c * chunk, chunk)],
                sem_s.at[b])

        idx_copy(0, 0).start()
        idx_copy(1, 1).start()

        def body(i, carry):
            for b in range(2):
                c = 2 * i + b

                @pl.when(c >= 2)
                def _wait_store():
                    store_copy(c - 2, b).wait()

                idx_copy(c, b).wait()
                gather_copy(b).start()
                gather_copy(b).wait()

                @pl.when(c + 2 < n_chunks)
                def _prefetch_idx():
                    idx_copy(c + 2, b).start()

                store_copy(c, b).start()
            return carry

        lax.fori_loop(0, n_chunks // 2, body, 0)
        store_copy(n_chunks - 2, 0).wait()
        store_copy(n_chunks - 1, 1).wait()

    return k(weight, flat_idx)


def kernel(idx, weight):
    S0, S1 = idx.shape
    D = weight.shape[1]
    flat = idx.reshape(S0 * S1).astype(jnp.int32)
    out = _gather(weight, flat, 1600)
    return out.reshape(S0, S1, D)


# speedup vs baseline: 1.5865x; 1.5865x over previous
import functools

import jax
import jax.numpy as jnp
from jax import lax
from jax.experimental import pallas as pl
from jax.experimental.pallas import tpu as pltpu
from jax.experimental.pallas import tpu_sc as plsc

_NC = 2
_NS = 16
_NW = _NC * _NS


@functools.partial(jax.jit, static_argnums=(2,))
def _gather(weight, idx, rows_per_chunk):
    R, S = idx.shape
    D = weight.shape[1]
    rows_per_w = R // _NW
    n_chunks = rows_per_w // rows_per_chunk
    assert n_chunks % 2 == 0 and n_chunks * rows_per_chunk == rows_per_w
    mesh = plsc.VectorSubcoreMesh(core_axis_name="c", subcore_axis_name="s")

    @functools.partial(
        pl.kernel,
        mesh=mesh,
        out_type=jax.ShapeDtypeStruct((R, S, D), jnp.float32),
        scratch_types=[
            pltpu.VMEM((2, rows_per_chunk, S), jnp.int32),
            pltpu.VMEM((2, rows_per_chunk, S, D), jnp.float32),
            pltpu.SemaphoreType.DMA((2,)),
            pltpu.SemaphoreType.DMA((2,)),
            pltpu.SemaphoreType.DMA((2,)),
        ],
        compiler_params=pltpu.CompilerParams(use_tc_tiling_on_sc=False),
    )
    def k(table_hbm, idx_hbm, out_hbm, idx_v, rows_v, sem_i, sem_g, sem_s):
        wid = lax.axis_index("s") * _NC + lax.axis_index("c")
        base = wid * rows_per_w

        def idx_copy(c, b):
            return pltpu.make_async_copy(
                idx_hbm.at[pl.ds(base + c * rows_per_chunk, rows_per_chunk)],
                idx_v.at[b], sem_i.at[b])

        def gather_copy(b, r):
            return pltpu.make_async_copy(
                table_hbm.at[idx_v.at[b, r]], rows_v.at[b, r], sem_g.at[b])

        def store_copy(c, b):
            return pltpu.make_async_copy(
                rows_v.at[b],
                out_hbm.at[pl.ds(base + c * rows_per_chunk, rows_per_chunk)],
                sem_s.at[b])

        idx_copy(0, 0).start()
        idx_copy(1, 1).start()

        def body(i, carry):
            for b in range(2):
                c = 2 * i + b

                @pl.when(c >= 2)
                def _wait_store():
                    store_copy(c - 2, b).wait()

                idx_copy(c, b).wait()
                for r in range(rows_per_chunk):
                    gather_copy(b, r).start()
                for r in range(rows_per_chunk):
                    gather_copy(b, r).wait()

                @pl.when(c + 2 < n_chunks)
                def _prefetch_idx():
                    idx_copy(c + 2, b).start()

                store_copy(c, b).start()
            return carry

        lax.fori_loop(0, n_chunks // 2, body, 0)
        store_copy(n_chunks - 2, 0).wait()
        store_copy(n_chunks - 1, 1).wait()

    return k(weight, idx)


def kernel(idx, weight):
    return _gather(weight, idx.astype(jnp.int32), 8)
